# baseline (speedup 1.0000x reference)
import functools

import jax
import jax.numpy as jnp
from jax import lax
from jax.experimental import pallas as pl
from jax.experimental.pallas import tpu as pltpu
from jax.experimental.pallas import tpu_sc as plsc

_B = 64
_C = 384
_ROWS_PER_WORKER = 8
_NUM_ACTIVE = 16


def _sc_gather(y, gammas_table, betas_table):
    mesh = plsc.VectorSubcoreMesh(core_axis_name="c", subcore_axis_name="s")

    @functools.partial(
        pl.kernel,
        out_type=[
            jax.ShapeDtypeStruct((_B, _C), jnp.float32),
            jax.ShapeDtypeStruct((_B, _C), jnp.float32),
        ],
        mesh=mesh,
        scratch_types=[
            pltpu.VMEM((_ROWS_PER_WORKER,), jnp.int32),
            pltpu.VMEM((_ROWS_PER_WORKER, _C), jnp.float32),
            pltpu.SemaphoreType.DMA,
        ],
    )
    def gather_kernel(y_hbm, gt_hbm, bt_hbm, g_out, b_out, idx_v, rows_v, sem):
        wid = lax.axis_index("s") * 2 + lax.axis_index("c")
        base = (wid % 8) * _ROWS_PER_WORKER

        @pl.when(wid < 8)
        def _():
            pltpu.sync_copy(y_hbm.at[pl.ds(base, _ROWS_PER_WORKER)], idx_v)
            pltpu.async_copy(gt_hbm.at[idx_v], rows_v, sem).wait()
            pltpu.sync_copy(rows_v, g_out.at[pl.ds(base, _ROWS_PER_WORKER)])

        @pl.when((wid >= 8) & (wid < _NUM_ACTIVE))
        def _():
            pltpu.sync_copy(y_hbm.at[pl.ds(base, _ROWS_PER_WORKER)], idx_v)
            pltpu.async_copy(bt_hbm.at[idx_v], rows_v, sem).wait()
            pltpu.sync_copy(rows_v, b_out.at[pl.ds(base, _ROWS_PER_WORKER)])

    return gather_kernel(y, gammas_table, betas_table)


_PB = 28
_NBUF = 4


def _film_ring_body(xt_ref, g_ref, b_ref, o_ref, xbuf, obuf, insem, outsem):
    p = xt_ref.shape[0]
    nchunks = p // _PB
    g = g_ref[...]
    b = b_ref[...]

    for k in range(_NBUF):
        pltpu.make_async_copy(
            xt_ref.at[pl.ds(k * _PB, _PB)], xbuf.at[k], insem.at[k]
        ).start()

    for j in range(nchunks):
        slot = j % _NBUF
        pltpu.make_async_copy(
            xt_ref.at[pl.ds(j * _PB, _PB)], xbuf.at[slot], insem.at[slot]
        ).wait()

        if j >= _NBUF:
            pltpu.make_async_copy(
                obuf.at[slot], o_ref.at[pl.ds(0, _PB)], outsem.at[slot]
            ).wait()

        obuf[slot] = xbuf[slot] * g + b

        pltpu.make_async_copy(
            obuf.at[slot], o_ref.at[pl.ds(j * _PB, _PB)], outsem.at[slot]
        ).start()

        if j + _NBUF < nchunks:
            pltpu.make_async_copy(
                xt_ref.at[pl.ds((j + _NBUF) * _PB, _PB)],
                xbuf.at[slot],
                insem.at[slot],
            ).start()

    for k in range(_NBUF):
        pltpu.make_async_copy(
            obuf.at[k], o_ref.at[pl.ds(0, _PB)], outsem.at[k]
        ).wait()


def _film_planes(xt, g, b):
    p, bsz, c = xt.shape
    return pl.pallas_call(
        _film_ring_body,
        in_specs=[
            pl.BlockSpec(memory_space=pltpu.HBM),
            pl.BlockSpec(memory_space=pltpu.VMEM),
            pl.BlockSpec(memory_space=pltpu.VMEM),
        ],
        out_specs=pl.BlockSpec(memory_space=pltpu.HBM),
        out_shape=jax.ShapeDtypeStruct((p, bsz, c), xt.dtype),
        scratch_shapes=[
            pltpu.VMEM((_NBUF, _PB, bsz, c), jnp.float32),
            pltpu.VMEM((_NBUF, _PB, bsz, c), jnp.float32),
            pltpu.SemaphoreType.DMA((_NBUF,)),
            pltpu.SemaphoreType.DMA((_NBUF,)),
        ],
        compiler_params=pltpu.CompilerParams(
            vmem_limit_bytes=100 * 1024 * 1024,
        ),
    )(xt, g, b)


def kernel(x, y, gammas_table, betas_table):
    g, b = _sc_gather(y.astype(jnp.int32), gammas_table, betas_table)
    bsz, c, h, w = x.shape
    xt = jnp.transpose(x, (2, 3, 0, 1)).reshape(h * w, bsz, c)
    ot = _film_planes(xt, g, b)
    out = jnp.transpose(ot.reshape(h, w, bsz, c), (2, 3, 0, 1))
    return (out, y)


# speedup vs baseline: 1.0016x; 1.0016x over previous
import functools

import jax
import jax.numpy as jnp
from jax import lax
from jax.experimental import pallas as pl
from jax.experimental.pallas import tpu as pltpu
from jax.experimental.pallas import tpu_sc as plsc

_B = 64
_C = 384
_ROWS_PER_WORKER = 8
_NUM_ACTIVE = 16


def _sc_gather(y, gammas_table, betas_table):
    mesh = plsc.VectorSubcoreMesh(core_axis_name="c", subcore_axis_name="s")

    @functools.partial(
        pl.kernel,
        out_type=[
            jax.ShapeDtypeStruct((_B, _C), jnp.float32),
            jax.ShapeDtypeStruct((_B, _C), jnp.float32),
        ],
        mesh=mesh,
        scratch_types=[
            pltpu.VMEM((_ROWS_PER_WORKER,), jnp.int32),
            pltpu.VMEM((_ROWS_PER_WORKER, _C), jnp.float32),
            pltpu.SemaphoreType.DMA,
        ],
    )
    def gather_kernel(y_hbm, gt_hbm, bt_hbm, g_out, b_out, idx_v, rows_v, sem):
        wid = lax.axis_index("s") * 2 + lax.axis_index("c")
        base = (wid % 8) * _ROWS_PER_WORKER

        @pl.when(wid < 8)
        def _():
            pltpu.sync_copy(y_hbm.at[pl.ds(base, _ROWS_PER_WORKER)], idx_v)
            pltpu.async_copy(gt_hbm.at[idx_v], rows_v, sem).wait()
            pltpu.sync_copy(rows_v, g_out.at[pl.ds(base, _ROWS_PER_WORKER)])

        @pl.when((wid >= 8) & (wid < _NUM_ACTIVE))
        def _():
            pltpu.sync_copy(y_hbm.at[pl.ds(base, _ROWS_PER_WORKER)], idx_v)
            pltpu.async_copy(bt_hbm.at[idx_v], rows_v, sem).wait()
            pltpu.sync_copy(rows_v, b_out.at[pl.ds(base, _ROWS_PER_WORKER)])

    return gather_kernel(y, gammas_table, betas_table)


_PB = 56
_NBUF = 4


def _film_ring_body(xt_ref, g_ref, b_ref, o_ref, xbuf, obuf, insem, outsem):
    p = xt_ref.shape[0]
    nchunks = p // _PB
    g = g_ref[...]
    b = b_ref[...]

    for k in range(_NBUF):
        pltpu.make_async_copy(
            xt_ref.at[pl.ds(k * _PB, _PB)], xbuf.at[k], insem.at[k]
        ).start()

    for j in range(nchunks):
        slot = j % _NBUF
        pltpu.make_async_copy(
            xt_ref.at[pl.ds(j * _PB, _PB)], xbuf.at[slot], insem.at[slot]
        ).wait()

        if j >= _NBUF:
            pltpu.make_async_copy(
                obuf.at[slot], o_ref.at[pl.ds(0, _PB)], outsem.at[slot]
            ).wait()

        obuf[slot] = xbuf[slot] * g + b

        pltpu.make_async_copy(
            obuf.at[slot], o_ref.at[pl.ds(j * _PB, _PB)], outsem.at[slot]
        ).start()

        if j + _NBUF < nchunks:
            pltpu.make_async_copy(
                xt_ref.at[pl.ds((j + _NBUF) * _PB, _PB)],
                xbuf.at[slot],
                insem.at[slot],
            ).start()

    for k in range(_NBUF):
        pltpu.make_async_copy(
            obuf.at[k], o_ref.at[pl.ds(0, _PB)], outsem.at[k]
        ).wait()


def _film_planes(xt, g, b):
    p, bsz, c = xt.shape
    return pl.pallas_call(
        _film_ring_body,
        in_specs=[
            pl.BlockSpec(memory_space=pltpu.HBM),
            pl.BlockSpec(memory_space=pltpu.VMEM),
            pl.BlockSpec(memory_space=pltpu.VMEM),
        ],
        out_specs=pl.BlockSpec(memory_space=pltpu.HBM),
        out_shape=jax.ShapeDtypeStruct((p, bsz, c), xt.dtype),
        scratch_shapes=[
            pltpu.VMEM((_NBUF, _PB, bsz, c), jnp.float32),
            pltpu.VMEM((_NBUF, _PB, bsz, c), jnp.float32),
            pltpu.SemaphoreType.DMA((_NBUF,)),
            pltpu.SemaphoreType.DMA((_NBUF,)),
        ],
        compiler_params=pltpu.CompilerParams(
            vmem_limit_bytes=100 * 1024 * 1024,
        ),
    )(xt, g, b)


def kernel(x, y, gammas_table, betas_table):
    g, b = _sc_gather(y.astype(jnp.int32), gammas_table, betas_table)
    bsz, c, h, w = x.shape
    xt = jnp.transpose(x, (2, 3, 0, 1)).reshape(h * w, bsz, c)
    ot = _film_planes(xt, g, b)
    out = jnp.transpose(ot.reshape(h, w, bsz, c), (2, 3, 0, 1))
    return (out, y)


# speedup vs baseline: 1.3978x; 1.3956x over previous
import jax
import jax.numpy as jnp
from jax.experimental import pallas as pl
from jax.experimental.pallas import tpu as pltpu

_B = 64
_C = 384
_PB = 56
_NBUF = 4


def _film_ring_body(y_ref, gt_ref, bt_ref, xt_ref, o_ref,
                    gvm, bvm, xbuf, obuf, insem, outsem):
    p = xt_ref.shape[0]
    nchunks = p // _PB

    for k in range(_NBUF):
        pltpu.make_async_copy(
            xt_ref.at[pl.ds(k * _PB, _PB)], xbuf.at[k], insem.at[k]
        ).start()

    for i in range(_B):
        row = y_ref[i]
        gvm[i, :] = gt_ref[row, :]
        bvm[i, :] = bt_ref[row, :]
    g = gvm[...]
    b = bvm[...]

    for j in range(nchunks):
        slot = j % _NBUF
        pltpu.make_async_copy(
            xt_ref.at[pl.ds(j * _PB, _PB)], xbuf.at[slot], insem.at[slot]
        ).wait()

        if j >= _NBUF:
            pltpu.make_async_copy(
                obuf.at[slot], o_ref.at[pl.ds(0, _PB)], outsem.at[slot]
            ).wait()

        obuf[slot] = xbuf[slot] * g + b

        pltpu.make_async_copy(
            obuf.at[slot], o_ref.at[pl.ds(j * _PB, _PB)], outsem.at[slot]
        ).start(priority=1)

        if j + _NBUF < nchunks:
            pltpu.make_async_copy(
                xt_ref.at[pl.ds((j + _NBUF) * _PB, _PB)],
                xbuf.at[slot],
                insem.at[slot],
            ).start()

    for k in range(_NBUF):
        pltpu.make_async_copy(
            obuf.at[k], o_ref.at[pl.ds(0, _PB)], outsem.at[k]
        ).wait()


def _film_planes(y, gt, bt, xt):
    p, bsz, c = xt.shape
    return pl.pallas_call(
        _film_ring_body,
        in_specs=[
            pl.BlockSpec(memory_space=pltpu.SMEM),
            pl.BlockSpec(memory_space=pltpu.VMEM),
            pl.BlockSpec(memory_space=pltpu.VMEM),
            pl.BlockSpec(memory_space=pltpu.HBM),
        ],
        out_specs=pl.BlockSpec(memory_space=pltpu.HBM),
        out_shape=jax.ShapeDtypeStruct((p, bsz, c), xt.dtype),
        scratch_shapes=[
            pltpu.VMEM((_B, _C), jnp.float32),
            pltpu.VMEM((_B, _C), jnp.float32),
            pltpu.VMEM((_NBUF, _PB, _B, _C), jnp.float32),
            pltpu.VMEM((_NBUF, _PB, _B, _C), jnp.float32),
            pltpu.SemaphoreType.DMA((_NBUF,)),
            pltpu.SemaphoreType.DMA((_NBUF,)),
        ],
        compiler_params=pltpu.CompilerParams(
            vmem_limit_bytes=60 * 1024 * 1024,
        ),
    )(y, gt, bt, xt)


def kernel(x, y, gammas_table, betas_table):
    bsz, c, h, w = x.shape
    xt = jnp.transpose(x, (2, 3, 0, 1)).reshape(h * w, bsz, c)
    ot = _film_planes(y.astype(jnp.int32), gammas_table, betas_table, xt)
    out = jnp.transpose(ot.reshape(h, w, bsz, c), (2, 3, 0, 1))
    return (out, y)


# speedup vs baseline: 1.4064x; 1.0061x over previous
import jax
import jax.numpy as jnp
from jax.experimental import pallas as pl
from jax.experimental.pallas import tpu as pltpu

_B = 64
_C = 384
_PB = 112
_NBUF = 2


def _film_ring_body(y_ref, gt_ref, bt_ref, xt_ref, o_ref,
                    gvm, bvm, xbuf, obuf, insem, outsem):
    p = xt_ref.shape[0]
    nchunks = p // _PB

    for k in range(_NBUF):
        pltpu.make_async_copy(
            xt_ref.at[pl.ds(k * _PB, _PB)], xbuf.at[k], insem.at[k]
        ).start()

    for i in range(_B):
        row = y_ref[i]
        gvm[i, :] = gt_ref[row, :]
        bvm[i, :] = bt_ref[row, :]
    g = gvm[...]
    b = bvm[...]

    for j in range(nchunks):
        slot = j % _NBUF
        pltpu.make_async_copy(
            xt_ref.at[pl.ds(j * _PB, _PB)], xbuf.at[slot], insem.at[slot]
        ).wait()

        if j >= _NBUF:
            pltpu.make_async_copy(
                obuf.at[slot], o_ref.at[pl.ds(0, _PB)], outsem.at[slot]
            ).wait()

        obuf[slot] = xbuf[slot] * g + b

        pltpu.make_async_copy(
            obuf.at[slot], o_ref.at[pl.ds(j * _PB, _PB)], outsem.at[slot]
        ).start(priority=1)

        if j + _NBUF < nchunks:
            pltpu.make_async_copy(
                xt_ref.at[pl.ds((j + _NBUF) * _PB, _PB)],
                xbuf.at[slot],
                insem.at[slot],
            ).start()

    for k in range(_NBUF):
        pltpu.make_async_copy(
            obuf.at[k], o_ref.at[pl.ds(0, _PB)], outsem.at[k]
        ).wait()


def _film_planes(y, gt, bt, xt):
    p, bsz, c = xt.shape
    return pl.pallas_call(
        _film_ring_body,
        in_specs=[
            pl.BlockSpec(memory_space=pltpu.SMEM),
            pl.BlockSpec(memory_space=pltpu.VMEM),
            pl.BlockSpec(memory_space=pltpu.VMEM),
            pl.BlockSpec(memory_space=pltpu.HBM),
        ],
        out_specs=pl.BlockSpec(memory_space=pltpu.HBM),
        out_shape=jax.ShapeDtypeStruct((p, bsz, c), xt.dtype),
        scratch_shapes=[
            pltpu.VMEM((_B, _C), jnp.float32),
            pltpu.VMEM((_B, _C), jnp.float32),
            pltpu.VMEM((_NBUF, _PB, _B, _C), jnp.float32),
            pltpu.VMEM((_NBUF, _PB, _B, _C), jnp.float32),
            pltpu.SemaphoreType.DMA((_NBUF,)),
            pltpu.SemaphoreType.DMA((_NBUF,)),
        ],
        compiler_params=pltpu.CompilerParams(
            vmem_limit_bytes=60 * 1024 * 1024,
        ),
    )(y, gt, bt, xt)


def kernel(x, y, gammas_table, betas_table):
    bsz, c, h, w = x.shape
    xt = jnp.transpose(x, (2, 3, 0, 1)).reshape(h * w, bsz, c)
    ot = _film_planes(y.astype(jnp.int32), gammas_table, betas_table, xt)
    out = jnp.transpose(ot.reshape(h, w, bsz, c), (2, 3, 0, 1))
    return (out, y)


# speedup vs baseline: 1.4113x; 1.0035x over previous
import jax
import jax.numpy as jnp
from jax.experimental import pallas as pl
from jax.experimental.pallas import tpu as pltpu

_B = 64
_C = 384
_PB = 98
_NBUF = 3


def _film_ring_body(y_ref, gt_ref, bt_ref, xt_ref, o_ref,
                    gvm, bvm, xbuf, obuf, insem, outsem):
    p = xt_ref.shape[0]
    nchunks = p // _PB

    for k in range(_NBUF):
        pltpu.make_async_copy(
            xt_ref.at[pl.ds(k * _PB, _PB)], xbuf.at[k], insem.at[k]
        ).start()

    for i in range(_B):
        row = y_ref[i]
        gvm[i, :] = gt_ref[row, :]
        bvm[i, :] = bt_ref[row, :]
    g = gvm[...]
    b = bvm[...]

    for j in range(nchunks):
        slot = j % _NBUF
        pltpu.make_async_copy(
            xt_ref.at[pl.ds(j * _PB, _PB)], xbuf.at[slot], insem.at[slot]
        ).wait()

        if j >= _NBUF:
            pltpu.make_async_copy(
                obuf.at[slot], o_ref.at[pl.ds(0, _PB)], outsem.at[slot]
            ).wait()

        obuf[slot] = xbuf[slot] * g + b

        pltpu.make_async_copy(
            obuf.at[slot], o_ref.at[pl.ds(j * _PB, _PB)], outsem.at[slot]
        ).start(priority=1)

        if j + _NBUF < nchunks:
            pltpu.make_async_copy(
                xt_ref.at[pl.ds((j + _NBUF) * _PB, _PB)],
                xbuf.at[slot],
                insem.at[slot],
            ).start()

    for k in range(_NBUF):
        pltpu.make_async_copy(
            obuf.at[k], o_ref.at[pl.ds(0, _PB)], outsem.at[k]
        ).wait()


def _film_planes(y, gt, bt, xt):
    p, bsz, c = xt.shape
    return pl.pallas_call(
        _film_ring_body,
        in_specs=[
            pl.BlockSpec(memory_space=pltpu.SMEM),
            pl.BlockSpec(memory_space=pltpu.VMEM),
            pl.BlockSpec(memory_space=pltpu.VMEM),
            pl.BlockSpec(memory_space=pltpu.HBM),
        ],
        out_specs=pl.BlockSpec(memory_space=pltpu.HBM),
        out_shape=jax.ShapeDtypeStruct((p, bsz, c), xt.dtype),
        scratch_shapes=[
            pltpu.VMEM((_B, _C), jnp.float32),
            pltpu.VMEM((_B, _C), jnp.float32),
            pltpu.VMEM((_NBUF, _PB, _B, _C), jnp.float32),
            pltpu.VMEM((_NBUF, _PB, _B, _C), jnp.float32),
            pltpu.SemaphoreType.DMA((_NBUF,)),
            pltpu.SemaphoreType.DMA((_NBUF,)),
        ],
        compiler_params=pltpu.CompilerParams(
            vmem_limit_bytes=60 * 1024 * 1024,
        ),
    )(y, gt, bt, xt)


def kernel(x, y, gammas_table, betas_table):
    bsz, c, h, w = x.shape
    xt = jnp.transpose(x, (2, 3, 0, 1)).reshape(h * w, bsz, c)
    ot = _film_planes(y.astype(jnp.int32), gammas_table, betas_table, xt)
    out = jnp.transpose(ot.reshape(h, w, bsz, c), (2, 3, 0, 1))
    return (out, y)
